# baseline (speedup 1.0000x reference)
import functools

import jax
import jax.numpy as jnp
from jax import lax
from jax.experimental import pallas as pl
from jax.experimental.pallas import tpu as pltpu
from jax.experimental.pallas import tpu_sc as plsc

N = 10000
E = 320000
G = 64
D = 128
L = 16
NC = 2
NS = 16
NW = NC * NS
CHUNK = 128
NPAD = 10240
ROWS_PER_TILE = NPAD // NS
EPT_CHUNKS = 80
EPAD = NW * CHUNK * EPT_CHUNKS
RB = 512


@functools.cache
def _make_sc_agg(compute_deg):
  mesh = plsc.VectorSubcoreMesh(core_axis_name="c", subcore_axis_name="s",
                                num_cores=NC, num_subcores=NS)
  out_type = [jax.ShapeDtypeStruct((NC, NPAD, D), jnp.float32)]
  scratch = [
      pltpu.VMEM((CHUNK,), jnp.int32),
      pltpu.VMEM((CHUNK,), jnp.int32),
      pltpu.VMEM((CHUNK, D), jnp.float32),
      pltpu.VMEM_SHARED((NPAD, D), jnp.float32),
      pltpu.SemaphoreType.DMA,
  ]
  if compute_deg:
    out_type.append(jax.ShapeDtypeStruct((NW, NPAD), jnp.float32))
    scratch.append(pltpu.VMEM((NPAD,), jnp.float32))

  def body(x_hbm, src_hbm, dst_hbm, z2d_hbm, *rest):
    if compute_deg:
      z1d_hbm, part_hbm, deg_hbm, sidx, didx, rows0, acc, semR0, deg_v = rest
    else:
      part_hbm, sidx, didx, rows0, acc, semR0 = rest
    c = lax.axis_index("c")
    s = lax.axis_index("s")
    rbase = s * ROWS_PER_TILE
    wid = s * NC + c
    ebase = wid * (EPAD // NW)

    pltpu.sync_copy(z2d_hbm, acc.at[pl.ds(rbase, ROWS_PER_TILE)])
    if compute_deg:
      pltpu.sync_copy(z1d_hbm, deg_v)

    plsc.subcore_barrier()

    def hist(didx_row):
      for j in range(CHUNK // L):
        idx16 = didx_row[pl.ds(j * L, L)]
        counts, last = plsc.scan_count(idx16)
        plsc.addupdate_scatter(deg_v, [idx16], counts.astype(jnp.float32),
                               mask=last)

    def step(t, carry):
      off = ebase + t * CHUNK
      pltpu.sync_copy(src_hbm.at[pl.ds(off, CHUNK)], sidx)
      pltpu.sync_copy(dst_hbm.at[pl.ds(off, CHUNK)], didx)
      pltpu.async_copy(x_hbm.at[sidx], rows0, semR0).wait()
      pltpu.sync_copy(rows0, acc.at[didx], add=True)
      if compute_deg:
        hist(didx)
      return carry
    lax.fori_loop(0, EPT_CHUNKS, step, None)

    plsc.subcore_barrier()

    pltpu.sync_copy(acc.at[pl.ds(rbase, ROWS_PER_TILE)],
                    part_hbm.at[c].at[pl.ds(rbase, ROWS_PER_TILE)])
    if compute_deg:
      pltpu.sync_copy(deg_v, deg_hbm.at[wid])

  params = pltpu.CompilerParams(needs_layout_passes=False) if compute_deg else None
  return pl.kernel(body, out_type=tuple(out_type), mesh=mesh,
                   scratch_types=tuple(scratch), compiler_params=params)


def _combine(part, degp, xin, Wl, Wr, b):
  def body(p_ref, d_ref, x_ref, wl_ref, wr_ref, b_ref, o_ref):
    agg = p_ref[0] + p_ref[1]
    deg = jnp.sum(d_ref[...], axis=0)
    mean = agg * (1.0 / jnp.maximum(deg, 1.0))[:, None]
    o_ref[...] = (
        jnp.dot(mean, wl_ref[...], preferred_element_type=jnp.float32)
        + jnp.dot(x_ref[...], wr_ref[...], preferred_element_type=jnp.float32)
        + b_ref[...])
  return pl.pallas_call(
      body,
      grid=(NPAD // RB,),
      in_specs=[
          pl.BlockSpec((NC, RB, D), lambda i: (0, i, 0)),
          pl.BlockSpec((NW, RB), lambda i: (0, i)),
          pl.BlockSpec((RB, D), lambda i: (i, 0)),
          pl.BlockSpec((D, D), lambda i: (0, 0)),
          pl.BlockSpec((D, D), lambda i: (0, 0)),
          pl.BlockSpec((1, D), lambda i: (0, 0)),
      ],
      out_specs=pl.BlockSpec((RB, D), lambda i: (i, 0)),
      out_shape=jax.ShapeDtypeStruct((NPAD, D), jnp.float32),
  )(part, degp, xin, Wl, Wr, b)


def _combine_pool(part, degp, hin, Wl, Wr, b, batch3d):
  def body(p_ref, d_ref, h_ref, wl_ref, wr_ref, b_ref, bat_ref, o_ref):
    i = pl.program_id(0)
    @pl.when(i == 0)
    def _init():
      o_ref[...] = jnp.zeros_like(o_ref)
    agg = p_ref[0] + p_ref[1]
    deg = jnp.sum(d_ref[...], axis=0)
    mean = agg * (1.0 / jnp.maximum(deg, 1.0))[:, None]
    h2 = (jnp.dot(mean, wl_ref[...], preferred_element_type=jnp.float32)
          + jnp.dot(h_ref[...], wr_ref[...], preferred_element_type=jnp.float32)
          + b_ref[...])
    bat = bat_ref[0, 0, :]
    onehot = (bat[:, None] == lax.broadcasted_iota(jnp.int32, (RB, G), 1)
              ).astype(jnp.float32)
    o_ref[...] += lax.dot_general(onehot, h2, (((0,), (0,)), ((), ())),
                                  preferred_element_type=jnp.float32)
  return pl.pallas_call(
      body,
      grid=(NPAD // RB,),
      in_specs=[
          pl.BlockSpec((NC, RB, D), lambda i: (0, i, 0)),
          pl.BlockSpec((NW, RB), lambda i: (0, i)),
          pl.BlockSpec((RB, D), lambda i: (i, 0)),
          pl.BlockSpec((D, D), lambda i: (0, 0)),
          pl.BlockSpec((D, D), lambda i: (0, 0)),
          pl.BlockSpec((1, D), lambda i: (0, 0)),
          pl.BlockSpec((1, 1, RB), lambda i: (i, 0, 0)),
      ],
      out_specs=pl.BlockSpec((G, D), lambda i: (0, 0)),
      out_shape=jax.ShapeDtypeStruct((G, D), jnp.float32),
  )(part, degp, hin, Wl, Wr, b, batch3d)


def _head(pooled, W1, bl1, gamma, beta, rm, rv, W2p, bl2p):
  def body(p_ref, w1_ref, b1_ref, g_ref, be_ref, rm_ref, rv_ref, w2_ref,
           b2_ref, o_ref):
    h = jnp.dot(p_ref[...], w1_ref[...],
                preferred_element_type=jnp.float32) + b1_ref[...]
    h = (h - rm_ref[...]) * (g_ref[...] * lax.rsqrt(rv_ref[...] + 1e-5)) \
        + be_ref[...]
    h = jnp.maximum(h, 0.0)
    o_ref[...] = jnp.dot(h, w2_ref[...],
                         preferred_element_type=jnp.float32) + b2_ref[...]
  return pl.pallas_call(
      body,
      out_shape=jax.ShapeDtypeStruct((G, D), jnp.float32),
  )(pooled, W1, bl1, gamma, beta, rm, rv, W2p, bl2p)


def kernel(x, edge_index, batch, Wl1, Wr1, b1, Wl2, Wr2, b2, W1, bl1,
           gamma, beta, rm, rv, W2, bl2):
  src = edge_index[0]
  dst = edge_index[1]
  pad_e = EPAD - E
  src_p = jnp.concatenate([src, jnp.zeros((pad_e,), jnp.int32)])
  dst_p = jnp.concatenate([dst, jnp.full((pad_e,), N, jnp.int32)])
  x_pad = jnp.pad(x, ((0, NPAD - N), (0, 0)))
  batch3d = jnp.concatenate(
      [batch, jnp.full((NPAD - N,), G, jnp.int32)]).reshape(NPAD // RB, 1, RB)

  z2d = jnp.zeros((ROWS_PER_TILE, D), jnp.float32)
  z1d = jnp.zeros((NPAD,), jnp.float32)
  part1, degp = _make_sc_agg(True)(x_pad, src_p, dst_p, z2d, z1d)
  h = _combine(part1, degp, x_pad, Wl1, Wr1, b1.reshape(1, D))
  (part2,) = _make_sc_agg(False)(h, src_p, dst_p, z2d)
  pooled = _combine_pool(part2, degp, h, Wl2, Wr2, b2.reshape(1, D), batch3d)

  W2p = jnp.pad(W2, ((0, 0), (0, D - 2)))
  bl2p = jnp.pad(bl2, (0, D - 2)).reshape(1, D)
  out = _head(pooled, W1, bl1.reshape(1, D), gamma.reshape(1, D),
              beta.reshape(1, D), rm.reshape(1, D), rv.reshape(1, D),
              W2p, bl2p)
  return out[:, :2]


# speedup vs baseline: 1.2702x; 1.2702x over previous
import functools

import jax
import jax.numpy as jnp
from jax import lax
from jax.experimental import pallas as pl
from jax.experimental.pallas import tpu as pltpu
from jax.experimental.pallas import tpu_sc as plsc

N = 10000
E = 320000
G = 64
D = 128
L = 16
NC = 2
NS = 16
NW = NC * NS
CHUNK = 128
NPAD = 10240
ROWS_PER_TILE = NPAD // NS
EPT_CHUNKS = 79
EPAD = NW * CHUNK * EPT_CHUNKS
RB = 512


@functools.cache
def _make_sc_agg(compute_deg):
  mesh = plsc.VectorSubcoreMesh(core_axis_name="c", subcore_axis_name="s",
                                num_cores=NC, num_subcores=NS)
  out_type = [jax.ShapeDtypeStruct((NC, NPAD, D), jnp.float32)]
  scratch = [
      pltpu.VMEM((CHUNK,), jnp.int32),
      pltpu.VMEM((CHUNK,), jnp.int32),
      pltpu.VMEM((CHUNK, D), jnp.float32),
      pltpu.VMEM_SHARED((NPAD, D), jnp.float32),
      pltpu.SemaphoreType.DMA,
  ]
  if compute_deg:
    out_type.append(jax.ShapeDtypeStruct((NW, NPAD), jnp.float32))
    scratch.append(pltpu.VMEM((NPAD,), jnp.float32))

  def body(x_hbm, src_hbm, dst_hbm, z2d_hbm, *rest):
    if compute_deg:
      z1d_hbm, part_hbm, deg_hbm, sidx, didx, rows0, acc, semR0, deg_v = rest
    else:
      part_hbm, sidx, didx, rows0, acc, semR0 = rest
    c = lax.axis_index("c")
    s = lax.axis_index("s")
    rbase = s * ROWS_PER_TILE
    wid = s * NC + c
    ebase = wid * (EPAD // NW)

    pltpu.sync_copy(z2d_hbm, acc.at[pl.ds(rbase, ROWS_PER_TILE)])
    if compute_deg:
      pltpu.sync_copy(z1d_hbm, deg_v)

    plsc.subcore_barrier()

    def hist(didx_row):
      for j in range(CHUNK // L):
        idx16 = didx_row[pl.ds(j * L, L)]
        counts, last = plsc.scan_count(idx16)
        plsc.addupdate_scatter(deg_v, [idx16], counts.astype(jnp.float32),
                               mask=last)

    def step(t, carry):
      off = ebase + t * CHUNK
      pltpu.sync_copy(src_hbm.at[pl.ds(off, CHUNK)], sidx)
      pltpu.sync_copy(dst_hbm.at[pl.ds(off, CHUNK)], didx)
      pltpu.async_copy(x_hbm.at[sidx], rows0, semR0).wait()
      pltpu.sync_copy(rows0, acc.at[didx], add=True)
      if compute_deg:
        hist(didx)
      return carry
    lax.fori_loop(0, EPT_CHUNKS, step, None)

    plsc.subcore_barrier()

    pltpu.sync_copy(acc.at[pl.ds(rbase, ROWS_PER_TILE)],
                    part_hbm.at[c].at[pl.ds(rbase, ROWS_PER_TILE)])
    if compute_deg:
      pltpu.sync_copy(deg_v, deg_hbm.at[wid])

  params = pltpu.CompilerParams(needs_layout_passes=False) if compute_deg else None
  return pl.kernel(body, out_type=tuple(out_type), mesh=mesh,
                   scratch_types=tuple(scratch), compiler_params=params)


def _combine(part, degp, xin, Wl, Wr, b):
  def body(p_ref, d_ref, x_ref, wl_ref, wr_ref, b_ref, o_ref):
    agg = p_ref[0] + p_ref[1]
    deg = jnp.sum(d_ref[...], axis=0)
    mean = agg * (1.0 / jnp.maximum(deg, 1.0))[:, None]
    o_ref[...] = (
        jnp.dot(mean, wl_ref[...], preferred_element_type=jnp.float32)
        + jnp.dot(x_ref[...], wr_ref[...], preferred_element_type=jnp.float32)
        + b_ref[...])
  return pl.pallas_call(
      body,
      grid=(NPAD // RB,),
      in_specs=[
          pl.BlockSpec((NC, RB, D), lambda i: (0, i, 0)),
          pl.BlockSpec((NW, RB), lambda i: (0, i)),
          pl.BlockSpec((RB, D), lambda i: (i, 0)),
          pl.BlockSpec((D, D), lambda i: (0, 0)),
          pl.BlockSpec((D, D), lambda i: (0, 0)),
          pl.BlockSpec((1, D), lambda i: (0, 0)),
      ],
      out_specs=pl.BlockSpec((RB, D), lambda i: (i, 0)),
      out_shape=jax.ShapeDtypeStruct((NPAD, D), jnp.float32),
  )(part, degp, xin, Wl, Wr, b)


def _combine_pool(part, degp, hin, Wl, Wr, b, batch3d):
  def body(p_ref, d_ref, h_ref, wl_ref, wr_ref, b_ref, bat_ref, o_ref):
    i = pl.program_id(0)
    @pl.when(i == 0)
    def _init():
      o_ref[...] = jnp.zeros_like(o_ref)
    agg = p_ref[0] + p_ref[1]
    deg = jnp.sum(d_ref[...], axis=0)
    mean = agg * (1.0 / jnp.maximum(deg, 1.0))[:, None]
    h2 = (jnp.dot(mean, wl_ref[...], preferred_element_type=jnp.float32)
          + jnp.dot(h_ref[...], wr_ref[...], preferred_element_type=jnp.float32)
          + b_ref[...])
    bat = bat_ref[0, 0, :]
    onehot = (bat[:, None] == lax.broadcasted_iota(jnp.int32, (RB, G), 1)
              ).astype(jnp.float32)
    o_ref[...] += lax.dot_general(onehot, h2, (((0,), (0,)), ((), ())),
                                  preferred_element_type=jnp.float32)
  return pl.pallas_call(
      body,
      grid=(NPAD // RB,),
      in_specs=[
          pl.BlockSpec((NC, RB, D), lambda i: (0, i, 0)),
          pl.BlockSpec((NW, RB), lambda i: (0, i)),
          pl.BlockSpec((RB, D), lambda i: (i, 0)),
          pl.BlockSpec((D, D), lambda i: (0, 0)),
          pl.BlockSpec((D, D), lambda i: (0, 0)),
          pl.BlockSpec((1, D), lambda i: (0, 0)),
          pl.BlockSpec((1, 1, RB), lambda i: (i, 0, 0)),
      ],
      out_specs=pl.BlockSpec((G, D), lambda i: (0, 0)),
      out_shape=jax.ShapeDtypeStruct((G, D), jnp.float32),
  )(part, degp, hin, Wl, Wr, b, batch3d)


def _head(pooled, W1, bl1, gamma, beta, rm, rv, W2p, bl2p):
  def body(p_ref, w1_ref, b1_ref, g_ref, be_ref, rm_ref, rv_ref, w2_ref,
           b2_ref, o_ref):
    h = jnp.dot(p_ref[...], w1_ref[...],
                preferred_element_type=jnp.float32) + b1_ref[...]
    h = (h - rm_ref[...]) * (g_ref[...] * lax.rsqrt(rv_ref[...] + 1e-5)) \
        + be_ref[...]
    h = jnp.maximum(h, 0.0)
    o_ref[...] = jnp.dot(h, w2_ref[...],
                         preferred_element_type=jnp.float32) + b2_ref[...]
  return pl.pallas_call(
      body,
      out_shape=jax.ShapeDtypeStruct((G, D), jnp.float32),
  )(pooled, W1, bl1, gamma, beta, rm, rv, W2p, bl2p)


def kernel(x, edge_index, batch, Wl1, Wr1, b1, Wl2, Wr2, b2, W1, bl1,
           gamma, beta, rm, rv, W2, bl2):
  src = edge_index[0]
  dst = edge_index[1]
  pad_e = EPAD - E
  src_p = jnp.concatenate([src, jnp.zeros((pad_e,), jnp.int32)])
  dst_p = jnp.concatenate([dst, jnp.full((pad_e,), N, jnp.int32)])
  x_pad = jnp.pad(x, ((0, NPAD - N), (0, 0)))
  batch3d = jnp.concatenate(
      [batch, jnp.full((NPAD - N,), G, jnp.int32)]).reshape(NPAD // RB, 1, RB)

  z2d = jnp.zeros((ROWS_PER_TILE, D), jnp.float32)
  z1d = jnp.zeros((NPAD,), jnp.float32)
  part1, degp = _make_sc_agg(True)(x_pad, src_p, dst_p, z2d, z1d)
  h = _combine(part1, degp, x_pad, Wl1, Wr1, b1.reshape(1, D))
  (part2,) = _make_sc_agg(False)(h, src_p, dst_p, z2d)
  pooled = _combine_pool(part2, degp, h, Wl2, Wr2, b2.reshape(1, D), batch3d)

  W2p = jnp.pad(W2, ((0, 0), (0, D - 2)))
  bl2p = jnp.pad(bl2, (0, D - 2)).reshape(1, D)
  out = _head(pooled, W1, bl1.reshape(1, D), gamma.reshape(1, D),
              beta.reshape(1, D), rm.reshape(1, D), rv.reshape(1, D),
              W2p, bl2p)
  return out[:, :2]


# speedup vs baseline: 1.4537x; 1.1444x over previous
import functools

import jax
import jax.numpy as jnp
from jax import lax
from jax.experimental import pallas as pl
from jax.experimental.pallas import tpu as pltpu
from jax.experimental.pallas import tpu_sc as plsc

N = 10000
E = 320000
G = 64
D = 128
L = 16
NC = 2
NS = 16
NW = NC * NS
CHUNK = 128
NPAD = 10240
ROWS_PER_TILE = NPAD // NS
CT0 = 99
CT1 = 59
EPAD = NS * (CT0 + CT1) * CHUNK
RB = 512


@functools.cache
def _make_sc_agg(compute_deg):
  mesh = plsc.VectorSubcoreMesh(core_axis_name="c", subcore_axis_name="s",
                                num_cores=NC, num_subcores=NS)
  out_type = [jax.ShapeDtypeStruct((NC, NPAD, D), jnp.float32)]
  scratch = [
      pltpu.VMEM((CHUNK,), jnp.int32),
      pltpu.VMEM((CHUNK,), jnp.int32),
      pltpu.VMEM((CHUNK, D), jnp.float32),
      pltpu.VMEM_SHARED((NPAD, D), jnp.float32),
      pltpu.SemaphoreType.DMA,
  ]
  if compute_deg:
    out_type.append(jax.ShapeDtypeStruct((NW, NPAD), jnp.float32))
    scratch.append(pltpu.VMEM((NPAD,), jnp.float32))

  def body(x_hbm, src_hbm, dst_hbm, z2d_hbm, *rest):
    if compute_deg:
      z1d_hbm, part_hbm, deg_hbm, sidx, didx, rows0, acc, semR0, deg_v = rest
    else:
      part_hbm, sidx, didx, rows0, acc, semR0 = rest
    c = lax.axis_index("c")
    s = lax.axis_index("s")
    rbase = s * ROWS_PER_TILE
    wid = s * NC + c
    cbase = jnp.where(c == 0, s * CT0, NS * CT0 + s * CT1)
    nch = jnp.where(c == 0, CT0, CT1)

    pltpu.sync_copy(z2d_hbm, acc.at[pl.ds(rbase, ROWS_PER_TILE)])
    if compute_deg:
      pltpu.sync_copy(z1d_hbm, deg_v)

    plsc.subcore_barrier()

    def hist(didx_row):
      for j in range(CHUNK // L):
        idx16 = didx_row[pl.ds(j * L, L)]
        counts, last = plsc.scan_count(idx16)
        plsc.addupdate_scatter(deg_v, [idx16], counts.astype(jnp.float32),
                               mask=last)

    def step(t, carry):
      off = (cbase + t) * CHUNK
      pltpu.sync_copy(src_hbm.at[pl.ds(off, CHUNK)], sidx)
      pltpu.sync_copy(dst_hbm.at[pl.ds(off, CHUNK)], didx)
      pltpu.async_copy(x_hbm.at[sidx], rows0, semR0).wait()
      pltpu.sync_copy(rows0, acc.at[didx], add=True)
      if compute_deg:
        hist(didx)
      return carry
    lax.fori_loop(0, nch, step, None)

    plsc.subcore_barrier()

    pltpu.sync_copy(acc.at[pl.ds(rbase, ROWS_PER_TILE)],
                    part_hbm.at[c].at[pl.ds(rbase, ROWS_PER_TILE)])
    if compute_deg:
      pltpu.sync_copy(deg_v, deg_hbm.at[wid])

  params = pltpu.CompilerParams(needs_layout_passes=False) if compute_deg else None
  return pl.kernel(body, out_type=tuple(out_type), mesh=mesh,
                   scratch_types=tuple(scratch), compiler_params=params)


def _combine(part, degp, xin, Wl, Wr, b):
  def body(p_ref, d_ref, x_ref, wl_ref, wr_ref, b_ref, o_ref):
    agg = p_ref[0] + p_ref[1]
    deg = jnp.sum(d_ref[...], axis=0)
    mean = agg * (1.0 / jnp.maximum(deg, 1.0))[:, None]
    o_ref[...] = (
        jnp.dot(mean, wl_ref[...], preferred_element_type=jnp.float32)
        + jnp.dot(x_ref[...], wr_ref[...], preferred_element_type=jnp.float32)
        + b_ref[...])
  return pl.pallas_call(
      body,
      grid=(NPAD // RB,),
      in_specs=[
          pl.BlockSpec((NC, RB, D), lambda i: (0, i, 0)),
          pl.BlockSpec((NW, RB), lambda i: (0, i)),
          pl.BlockSpec((RB, D), lambda i: (i, 0)),
          pl.BlockSpec((D, D), lambda i: (0, 0)),
          pl.BlockSpec((D, D), lambda i: (0, 0)),
          pl.BlockSpec((1, D), lambda i: (0, 0)),
      ],
      out_specs=pl.BlockSpec((RB, D), lambda i: (i, 0)),
      out_shape=jax.ShapeDtypeStruct((NPAD, D), jnp.float32),
  )(part, degp, xin, Wl, Wr, b)


def _combine_pool(part, degp, hin, Wl, Wr, b, batch3d):
  def body(p_ref, d_ref, h_ref, wl_ref, wr_ref, b_ref, bat_ref, o_ref):
    i = pl.program_id(0)
    @pl.when(i == 0)
    def _init():
      o_ref[...] = jnp.zeros_like(o_ref)
    agg = p_ref[0] + p_ref[1]
    deg = jnp.sum(d_ref[...], axis=0)
    mean = agg * (1.0 / jnp.maximum(deg, 1.0))[:, None]
    h2 = (jnp.dot(mean, wl_ref[...], preferred_element_type=jnp.float32)
          + jnp.dot(h_ref[...], wr_ref[...], preferred_element_type=jnp.float32)
          + b_ref[...])
    bat = bat_ref[0, 0, :]
    onehot = (bat[:, None] == lax.broadcasted_iota(jnp.int32, (RB, G), 1)
              ).astype(jnp.float32)
    o_ref[...] += lax.dot_general(onehot, h2, (((0,), (0,)), ((), ())),
                                  preferred_element_type=jnp.float32)
  return pl.pallas_call(
      body,
      grid=(NPAD // RB,),
      in_specs=[
          pl.BlockSpec((NC, RB, D), lambda i: (0, i, 0)),
          pl.BlockSpec((NW, RB), lambda i: (0, i)),
          pl.BlockSpec((RB, D), lambda i: (i, 0)),
          pl.BlockSpec((D, D), lambda i: (0, 0)),
          pl.BlockSpec((D, D), lambda i: (0, 0)),
          pl.BlockSpec((1, D), lambda i: (0, 0)),
          pl.BlockSpec((1, 1, RB), lambda i: (i, 0, 0)),
      ],
      out_specs=pl.BlockSpec((G, D), lambda i: (0, 0)),
      out_shape=jax.ShapeDtypeStruct((G, D), jnp.float32),
  )(part, degp, hin, Wl, Wr, b, batch3d)


def _head(pooled, W1, bl1, gamma, beta, rm, rv, W2p, bl2p):
  def body(p_ref, w1_ref, b1_ref, g_ref, be_ref, rm_ref, rv_ref, w2_ref,
           b2_ref, o_ref):
    h = jnp.dot(p_ref[...], w1_ref[...],
                preferred_element_type=jnp.float32) + b1_ref[...]
    h = (h - rm_ref[...]) * (g_ref[...] * lax.rsqrt(rv_ref[...] + 1e-5)) \
        + be_ref[...]
    h = jnp.maximum(h, 0.0)
    o_ref[...] = jnp.dot(h, w2_ref[...],
                         preferred_element_type=jnp.float32) + b2_ref[...]
  return pl.pallas_call(
      body,
      out_shape=jax.ShapeDtypeStruct((G, D), jnp.float32),
  )(pooled, W1, bl1, gamma, beta, rm, rv, W2p, bl2p)


def kernel(x, edge_index, batch, Wl1, Wr1, b1, Wl2, Wr2, b2, W1, bl1,
           gamma, beta, rm, rv, W2, bl2):
  src = edge_index[0]
  dst = edge_index[1]
  pad_e = EPAD - E
  src_p = jnp.concatenate([src, jnp.zeros((pad_e,), jnp.int32)])
  dst_p = jnp.concatenate([dst, jnp.full((pad_e,), N, jnp.int32)])
  x_pad = jnp.pad(x, ((0, NPAD - N), (0, 0)))
  batch3d = jnp.concatenate(
      [batch, jnp.full((NPAD - N,), G, jnp.int32)]).reshape(NPAD // RB, 1, RB)

  z2d = jnp.zeros((ROWS_PER_TILE, D), jnp.float32)
  z1d = jnp.zeros((NPAD,), jnp.float32)
  part1, degp = _make_sc_agg(True)(x_pad, src_p, dst_p, z2d, z1d)
  h = _combine(part1, degp, x_pad, Wl1, Wr1, b1.reshape(1, D))
  (part2,) = _make_sc_agg(False)(h, src_p, dst_p, z2d)
  pooled = _combine_pool(part2, degp, h, Wl2, Wr2, b2.reshape(1, D), batch3d)

  W2p = jnp.pad(W2, ((0, 0), (0, D - 2)))
  bl2p = jnp.pad(bl2, (0, D - 2)).reshape(1, D)
  out = _head(pooled, W1, bl1.reshape(1, D), gamma.reshape(1, D),
              beta.reshape(1, D), rm.reshape(1, D), rv.reshape(1, D),
              W2p, bl2p)
  return out[:, :2]
